# baseline (speedup 1.0000x reference)
import functools

import jax
import jax.numpy as jnp
from jax import lax
from jax.experimental import pallas as pl
from jax.experimental.pallas import tpu as pltpu
from jax.experimental.pallas import tpu_sc as plsc

_HIDDEN = 128
_N = 100000
_NC = 2
_NS = 16
_NW = _NC * _NS
_RPW = _N // _NW
_B = 48
_NB = 65
_SLAB = 3128
_SLABREF = 3136
_NSETS = 4
_LANES = 16
_G = _B // _LANES
_TROWS = 513


def _bound(w):
    return pl.multiple_of(((w * _RPW + 4) >> 3) << 3, 8)


def _sc_body(x_hbm, ind_hbm, outd_hbm, int_hbm, outt_hbm, o_hbm,
             slab_in, slab_out, tin_res, rows_out, xbuf, sem_g, sem_s):
    wid = lax.axis_index("s") * _NC + lax.axis_index("c")
    s0 = _bound(wid)
    cnt = _bound(wid + 1) - s0

    pltpu.sync_copy(int_hbm, tin_res)
    pltpu.sync_copy(ind_hbm.at[pl.ds(s0, _SLAB)], slab_in.at[pl.ds(0, _SLAB)])
    pltpu.sync_copy(outd_hbm.at[pl.ds(s0, _SLAB)], slab_out.at[pl.ds(0, _SLAB)])

    def prep(m, s):
        pltpu.async_copy(
            outt_hbm.at[slab_out.at[pl.ds(m * _B, _B)]], rows_out[s], sem_g[s])
        pltpu.async_copy(x_hbm.at[pl.ds(s0 + m * _B, _B)], xbuf[s], sem_g[s])

    def wait_gathers(s):
        pltpu.make_async_copy(outt_hbm.at[slab_out.at[pl.ds(0, _B)]],
                              rows_out[s], sem_g[s]).wait()
        pltpu.make_async_copy(x_hbm.at[pl.ds(s0, _B)], xbuf[s], sem_g[s]).wait()

    def start_store(m, s):
        pltpu.async_copy(xbuf[s], o_hbm.at[pl.ds(s0 + m * _B, _B)], sem_s[s])

    def wait_store(s):
        pltpu.make_async_copy(xbuf[s], o_hbm.at[pl.ds(s0, _B)], sem_s[s]).wait()

    def add_rows(xb, ro, idx_off, ngroups, clamp=False):
        def grp(g, carry):
            idxv = slab_in[pl.ds(idx_off + g * _LANES, _LANES)]
            if clamp:
                idxv = jnp.minimum(jnp.maximum(idxv, 0), _TROWS - 1)
            base = g * _LANES
            for l in range(_LANES):
                ri = idxv[l]
                lr = base + l
                for j in range(_HIDDEN // _LANES):
                    sl = pl.ds(j * _LANES, _LANES)
                    xb[lr, sl] = xb[lr, sl] + tin_res[ri, sl] + ro[lr, sl]
            return carry

        lax.fori_loop(0, ngroups, grp, 0)

    prep(0, 0)
    prep(1, 1)

    def outer(t, carry):
        for j in range(_NSETS):
            k = t * _NSETS + j
            s = j
            s2 = (j + 2) % _NSETS

            @pl.when(k + 2 < _NB)
            def _():
                @pl.when(k >= 2)
                def _():
                    wait_store(s2)
                prep(k + 2, s2)

            @pl.when(k < _NB)
            def _():
                wait_gathers(s)
                add_rows(xbuf[s], rows_out[s], k * _B, _G)
                start_store(k, s)
        return carry

    lax.fori_loop(0, (_NB + _NSETS - 1) // _NSETS, outer, 0)

    for s in range(_NSETS):
        wait_store(s)

    @pl.when(cnt == _SLAB)
    def _():
        voff = _NB * _B
        pltpu.async_copy(
            outt_hbm.at[slab_out.at[pl.ds(voff, 8)]],
            rows_out[0].at[pl.ds(0, 8)], sem_g[0])
        pltpu.async_copy(
            x_hbm.at[pl.ds(s0 + voff, 8)], xbuf[0].at[pl.ds(0, 8)],
            sem_g[0])
        pltpu.make_async_copy(outt_hbm.at[slab_out.at[pl.ds(0, 8)]],
                              rows_out[0].at[pl.ds(0, 8)], sem_g[0]).wait()
        pltpu.make_async_copy(x_hbm.at[pl.ds(s0, 8)],
                              xbuf[0].at[pl.ds(0, 8)], sem_g[0]).wait()
        add_rows(xbuf[0], rows_out[0], voff, 1, clamp=True)
        pltpu.sync_copy(xbuf[0].at[pl.ds(0, 8)],
                        o_hbm.at[pl.ds(s0 + voff, 8)])


@functools.partial(
    pl.kernel,
    mesh=plsc.VectorSubcoreMesh(core_axis_name="c", subcore_axis_name="s"),
    out_type=jax.ShapeDtypeStruct((_N, _HIDDEN), jnp.float32),
    scratch_types=[
        pltpu.VMEM((_SLABREF,), jnp.int32),
        pltpu.VMEM((_SLABREF,), jnp.int32),
        pltpu.VMEM((_TROWS, _HIDDEN), jnp.float32),
        [pltpu.VMEM((_B, _HIDDEN), jnp.float32) for _ in range(_NSETS)],
        [pltpu.VMEM((_B, _HIDDEN), jnp.float32) for _ in range(_NSETS)],
        [pltpu.SemaphoreType.DMA for _ in range(_NSETS)],
        [pltpu.SemaphoreType.DMA for _ in range(_NSETS)],
    ],
)
def _centrality_sc(x, ind, outd, int_t, outt, o,
                   slab_in, slab_out, tin_res, rows_out, xbuf, sem_g, sem_s):
    _sc_body(x, ind, outd, int_t, outt, o,
             slab_in, slab_out, tin_res, rows_out, xbuf, sem_g, sem_s)


def kernel(x, in_degree, out_degree, in_table, out_table):
    return _centrality_sc(
        x,
        in_degree.astype(jnp.int32),
        out_degree.astype(jnp.int32),
        in_table,
        out_table,
    )


# speedup vs baseline: 1.4333x; 1.4333x over previous
import functools

import jax
import jax.numpy as jnp
from jax import lax
from jax.experimental import pallas as pl
from jax.experimental.pallas import tpu as pltpu
from jax.experimental.pallas import tpu_sc as plsc

_HIDDEN = 128
_N = 100000
_NC = 2
_NS = 16
_NW = _NC * _NS
_RPW = _N // _NW
_B = 80
_NB = 39
_SLAB = 3128
_NSETS = 4
_LANES = 16


def _bound(w):
    return pl.multiple_of(((w * _RPW + 4) >> 3) << 3, 8)


def _sc_body(x_hbm, ind_hbm, outd_hbm, int_hbm, outt_hbm, o_hbm,
             slab_in, slab_out, rows_in, rows_out, xbuf, sem_g, sem_s):
    wid = lax.axis_index("s") * _NC + lax.axis_index("c")
    s0 = _bound(wid)
    cnt = _bound(wid + 1) - s0

    pltpu.sync_copy(ind_hbm.at[pl.ds(s0, _SLAB)], slab_in)
    pltpu.sync_copy(outd_hbm.at[pl.ds(s0, _SLAB)], slab_out)

    def prep(m, s):
        pltpu.async_copy(
            int_hbm.at[slab_in.at[pl.ds(m * _B, _B)]], rows_in[s], sem_g[s])
        pltpu.async_copy(
            outt_hbm.at[slab_out.at[pl.ds(m * _B, _B)]], rows_out[s], sem_g[s])
        pltpu.async_copy(x_hbm.at[pl.ds(s0 + m * _B, _B)], xbuf[s], sem_g[s])

    def wait_gathers(s):
        pltpu.make_async_copy(int_hbm.at[slab_in.at[pl.ds(0, _B)]],
                              rows_in[s], sem_g[s]).wait()
        pltpu.make_async_copy(outt_hbm.at[slab_out.at[pl.ds(0, _B)]],
                              rows_out[s], sem_g[s]).wait()
        pltpu.make_async_copy(x_hbm.at[pl.ds(s0, _B)], xbuf[s], sem_g[s]).wait()

    def start_store(m, s):
        pltpu.async_copy(xbuf[s], o_hbm.at[pl.ds(s0 + m * _B, _B)], sem_s[s])

    def wait_store(s):
        pltpu.make_async_copy(xbuf[s], o_hbm.at[pl.ds(s0, _B)], sem_s[s]).wait()

    def add_block(s, nrows):
        xb, ri, ro = xbuf[s], rows_in[s], rows_out[s]

        def row_fn(i, c):
            for j in range(_HIDDEN // _LANES):
                sl = pl.ds(j * _LANES, _LANES)
                xb[i, sl] = xb[i, sl] + ri[i, sl] + ro[i, sl]
            return c

        lax.fori_loop(0, nrows, row_fn, 0)

    prep(0, 0)
    prep(1, 1)

    def outer(t, carry):
        for j in range(_NSETS):
            k = t * _NSETS + j
            s = j
            s2 = (j + 2) % _NSETS

            @pl.when(k + 2 < _NB)
            def _():
                @pl.when(k >= 2)
                def _():
                    wait_store(s2)
                prep(k + 2, s2)

            @pl.when(k < _NB)
            def _():
                wait_gathers(s)
                add_block(s, _B)
                start_store(k, s)
        return carry

    lax.fori_loop(0, (_NB + _NSETS - 1) // _NSETS, outer, 0)

    for s in range(_NSETS):
        wait_store(s)

    @pl.when(cnt == _SLAB)
    def _():
        voff = _NB * _B
        pltpu.async_copy(
            int_hbm.at[slab_in.at[pl.ds(voff, 8)]],
            rows_in[0].at[pl.ds(0, 8)], sem_g[0])
        pltpu.async_copy(
            outt_hbm.at[slab_out.at[pl.ds(voff, 8)]],
            rows_out[0].at[pl.ds(0, 8)], sem_g[0])
        pltpu.async_copy(
            x_hbm.at[pl.ds(s0 + voff, 8)], xbuf[0].at[pl.ds(0, 8)], sem_g[0])
        pltpu.make_async_copy(int_hbm.at[slab_in.at[pl.ds(0, 8)]],
                              rows_in[0].at[pl.ds(0, 8)], sem_g[0]).wait()
        pltpu.make_async_copy(outt_hbm.at[slab_out.at[pl.ds(0, 8)]],
                              rows_out[0].at[pl.ds(0, 8)], sem_g[0]).wait()
        pltpu.make_async_copy(x_hbm.at[pl.ds(s0, 8)],
                              xbuf[0].at[pl.ds(0, 8)], sem_g[0]).wait()
        add_block(0, 8)
        pltpu.sync_copy(xbuf[0].at[pl.ds(0, 8)],
                        o_hbm.at[pl.ds(s0 + voff, 8)])


@functools.partial(
    pl.kernel,
    mesh=plsc.VectorSubcoreMesh(core_axis_name="c", subcore_axis_name="s"),
    out_type=jax.ShapeDtypeStruct((_N, _HIDDEN), jnp.float32),
    scratch_types=[
        pltpu.VMEM((_SLAB,), jnp.int32),
        pltpu.VMEM((_SLAB,), jnp.int32),
        [pltpu.VMEM((_B, _HIDDEN), jnp.float32) for _ in range(_NSETS)],
        [pltpu.VMEM((_B, _HIDDEN), jnp.float32) for _ in range(_NSETS)],
        [pltpu.VMEM((_B, _HIDDEN), jnp.float32) for _ in range(_NSETS)],
        [pltpu.SemaphoreType.DMA for _ in range(_NSETS)],
        [pltpu.SemaphoreType.DMA for _ in range(_NSETS)],
    ],
)
def _centrality_sc(x, ind, outd, int_t, outt, o,
                   slab_in, slab_out, rows_in, rows_out, xbuf, sem_g, sem_s):
    _sc_body(x, ind, outd, int_t, outt, o,
             slab_in, slab_out, rows_in, rows_out, xbuf, sem_g, sem_s)


def kernel(x, in_degree, out_degree, in_table, out_table):
    return _centrality_sc(
        x,
        in_degree.astype(jnp.int32),
        out_degree.astype(jnp.int32),
        in_table,
        out_table,
    )


# speedup vs baseline: 1.4358x; 1.0017x over previous
import functools

import jax
import jax.numpy as jnp
from jax import lax
from jax.experimental import pallas as pl
from jax.experimental.pallas import tpu as pltpu
from jax.experimental.pallas import tpu_sc as plsc

_HIDDEN = 128
_N = 100000
_NC = 2
_NS = 16
_NW = _NC * _NS
_RPW = _N // _NW
_B = 80
_NB = 39
_SLAB = 3128
_NSETS = 4
_LANES = 16


def _bound(w):
    return pl.multiple_of(((w * _RPW + 4) >> 3) << 3, 8)


def _sc_body(x_hbm, ind_hbm, outd_hbm, int_hbm, outt_hbm, o_hbm,
             slab_in, slab_out, rows_in, rows_out, xbuf, sem_g, sem_s):
    wid = lax.axis_index("s") * _NC + lax.axis_index("c")
    s0 = _bound(wid)
    cnt = _bound(wid + 1) - s0

    pltpu.async_copy(ind_hbm.at[pl.ds(s0, _SLAB)], slab_in, sem_g[0])
    pltpu.async_copy(outd_hbm.at[pl.ds(s0, _SLAB)], slab_out, sem_g[1])
    pltpu.make_async_copy(ind_hbm.at[pl.ds(s0, _SLAB)], slab_in,
                          sem_g[0]).wait()
    pltpu.make_async_copy(outd_hbm.at[pl.ds(s0, _SLAB)], slab_out,
                          sem_g[1]).wait()

    def prep(m, s):
        pltpu.async_copy(
            int_hbm.at[slab_in.at[pl.ds(m * _B, _B)]], rows_in[s], sem_g[s])
        pltpu.async_copy(
            outt_hbm.at[slab_out.at[pl.ds(m * _B, _B)]], rows_out[s], sem_g[s])
        pltpu.async_copy(x_hbm.at[pl.ds(s0 + m * _B, _B)], xbuf[s], sem_g[s])

    def wait_gathers(s):
        pltpu.make_async_copy(int_hbm.at[slab_in.at[pl.ds(0, _B)]],
                              rows_in[s], sem_g[s]).wait()
        pltpu.make_async_copy(outt_hbm.at[slab_out.at[pl.ds(0, _B)]],
                              rows_out[s], sem_g[s]).wait()
        pltpu.make_async_copy(x_hbm.at[pl.ds(s0, _B)], xbuf[s], sem_g[s]).wait()

    def start_store(m, s):
        pltpu.async_copy(xbuf[s], o_hbm.at[pl.ds(s0 + m * _B, _B)], sem_s[s])

    def wait_store(s):
        pltpu.make_async_copy(xbuf[s], o_hbm.at[pl.ds(s0, _B)], sem_s[s]).wait()

    def add_block(s, nrows):
        xb, ri, ro = xbuf[s], rows_in[s], rows_out[s]

        def row_fn(i, c):
            for j in range(_HIDDEN // _LANES):
                sl = pl.ds(j * _LANES, _LANES)
                xb[i, sl] = xb[i, sl] + ri[i, sl] + ro[i, sl]
            return c

        lax.fori_loop(0, nrows, row_fn, 0)

    prep(0, 0)
    prep(1, 1)

    def outer(t, carry):
        for j in range(_NSETS):
            k = t * _NSETS + j
            s = j
            s2 = (j + 2) % _NSETS

            @pl.when(k + 2 < _NB)
            def _():
                @pl.when(k >= 2)
                def _():
                    wait_store(s2)
                prep(k + 2, s2)

            @pl.when(k < _NB)
            def _():
                wait_gathers(s)
                add_block(s, _B)
                start_store(k, s)
        return carry

    lax.fori_loop(0, (_NB + _NSETS - 1) // _NSETS, outer, 0)

    for s in range(_NSETS):
        wait_store(s)

    @pl.when(cnt == _SLAB)
    def _():
        voff = _NB * _B
        pltpu.async_copy(
            int_hbm.at[slab_in.at[pl.ds(voff, 8)]],
            rows_in[0].at[pl.ds(0, 8)], sem_g[0])
        pltpu.async_copy(
            outt_hbm.at[slab_out.at[pl.ds(voff, 8)]],
            rows_out[0].at[pl.ds(0, 8)], sem_g[0])
        pltpu.async_copy(
            x_hbm.at[pl.ds(s0 + voff, 8)], xbuf[0].at[pl.ds(0, 8)], sem_g[0])
        pltpu.make_async_copy(int_hbm.at[slab_in.at[pl.ds(0, 8)]],
                              rows_in[0].at[pl.ds(0, 8)], sem_g[0]).wait()
        pltpu.make_async_copy(outt_hbm.at[slab_out.at[pl.ds(0, 8)]],
                              rows_out[0].at[pl.ds(0, 8)], sem_g[0]).wait()
        pltpu.make_async_copy(x_hbm.at[pl.ds(s0, 8)],
                              xbuf[0].at[pl.ds(0, 8)], sem_g[0]).wait()
        add_block(0, 8)
        pltpu.sync_copy(xbuf[0].at[pl.ds(0, 8)],
                        o_hbm.at[pl.ds(s0 + voff, 8)])


@functools.partial(
    pl.kernel,
    mesh=plsc.VectorSubcoreMesh(core_axis_name="c", subcore_axis_name="s"),
    out_type=jax.ShapeDtypeStruct((_N, _HIDDEN), jnp.float32),
    scratch_types=[
        pltpu.VMEM((_SLAB,), jnp.int32),
        pltpu.VMEM((_SLAB,), jnp.int32),
        [pltpu.VMEM((_B, _HIDDEN), jnp.float32) for _ in range(_NSETS)],
        [pltpu.VMEM((_B, _HIDDEN), jnp.float32) for _ in range(_NSETS)],
        [pltpu.VMEM((_B, _HIDDEN), jnp.float32) for _ in range(_NSETS)],
        [pltpu.SemaphoreType.DMA for _ in range(_NSETS)],
        [pltpu.SemaphoreType.DMA for _ in range(_NSETS)],
    ],
)
def _centrality_sc(x, ind, outd, int_t, outt, o,
                   slab_in, slab_out, rows_in, rows_out, xbuf, sem_g, sem_s):
    _sc_body(x, ind, outd, int_t, outt, o,
             slab_in, slab_out, rows_in, rows_out, xbuf, sem_g, sem_s)


def kernel(x, in_degree, out_degree, in_table, out_table):
    return _centrality_sc(
        x,
        in_degree.astype(jnp.int32),
        out_degree.astype(jnp.int32),
        in_table,
        out_table,
    )
